# baseline (speedup 1.0000x reference)
import functools

import jax
import jax.numpy as jnp
from jax import lax
from jax.experimental import pallas as pl
from jax.experimental.pallas import tpu as pltpu
from jax.experimental.pallas import tpu_sc as plsc

E_TOTAL = 6_400_000
NSEG = 200
SEG_PAD = 256
NC, NS = 2, 16
NW = NC * NS
CHUNK = E_TOTAL // NW
BLK = 20_000
NBLK = CHUNK // BLK
SUB = 2_000
NSUB = BLK // SUB
VPS = SUB // 16
HVPS = (VPS - 1) // 2
UNROLL = 62
EPS = 1e-6
STD_W = 0.5
LN2 = 0.6931471805599453
SQRT2 = 1.41421356237


def _logf(y):
    i = lax.bitcast_convert_type(y, jnp.int32)
    k = lax.shift_right_arithmetic(i - jnp.int32(0x3F330000), 23)
    m = lax.bitcast_convert_type(i - lax.shift_left(k, 23), jnp.float32)
    ef = k.astype(jnp.float32)
    f = m - 1.0
    z = f * f
    p = jnp.float32(0.19074033)
    for c in (-0.26126555, 0.33371909):
        p = p * f + jnp.float32(c)
    r = z * (p * f - 0.5) + f
    r = r + ef * LN2
    return r


def _sc_partials(x, idx):
    mesh = plsc.VectorSubcoreMesh(core_axis_name="c", subcore_axis_name="s")

    @functools.partial(
        pl.kernel,
        out_type=jax.ShapeDtypeStruct((NW, 4, SEG_PAD), jnp.float32),
        mesh=mesh,
        scratch_types=[
            pltpu.VMEM((BLK,), jnp.float32),
            pltpu.VMEM((BLK,), jnp.int32),
            pltpu.VMEM((BLK,), jnp.float32),
            pltpu.VMEM((BLK,), jnp.int32),
            pltpu.VMEM((16 * SEG_PAD,), jnp.float32),
            pltpu.VMEM((16 * SEG_PAD,), jnp.float32),
            pltpu.VMEM((16 * SEG_PAD,), jnp.float32),
            pltpu.VMEM((16 * SEG_PAD,), jnp.float32),
            pltpu.VMEM((4, SEG_PAD), jnp.float32),
            pltpu.SemaphoreType.DMA,
            pltpu.SemaphoreType.DMA,
        ],
        compiler_params=pltpu.CompilerParams(needs_layout_passes=False),
    )
    def k(x_hbm, idx_hbm, out_hbm, xb0, ib0, xb1, ib1, ax, al, al2, ac, ob,
          sem0, sem1):
        cid = lax.axis_index("c")
        sid = lax.axis_index("s")
        wid = sid * NC + cid
        base = wid * CHUNK
        zeros = jnp.zeros((16,), jnp.float32)
        ones = jnp.ones((16,), jnp.float32)
        laneoff = lax.iota(jnp.int32, 16)

        @plsc.parallel_loop(0, 16 * SEG_PAD // 16, unroll=8)
        def zbody(g):
            ax[pl.ds(g * 16, 16)] = zeros
            al[pl.ds(g * 16, 16)] = zeros
            al2[pl.ds(g * 16, 16)] = zeros
            ac[pl.ds(g * 16, 16)] = zeros

        def start_blk(b, xbuf, ibuf, sem):
            o = base + b * BLK
            pltpu.make_async_copy(x_hbm.at[pl.ds(o, BLK)], xbuf, sem).start()
            pltpu.make_async_copy(idx_hbm.at[pl.ds(o, BLK)], ibuf, sem).start()

        def wait_blk(xbuf, ibuf, sem):
            pltpu.make_async_copy(x_hbm.at[pl.ds(0, BLK)], xbuf, sem).wait()
            pltpu.make_async_copy(idx_hbm.at[pl.ds(0, BLK)], ibuf, sem).wait()

        def process(xbuf, ibuf):
            def sbody(sb, c):
                soff = sb * SUB
                lo = jnp.min(ibuf[pl.ds(soff, 16)])
                hi = jnp.max(ibuf[pl.ds(soff + SUB - 16, 16)])

                @pl.when(lo == hi)
                def _():
                    def one(off):
                        xv = xbuf[pl.ds(off, 16)]
                        y = jnp.abs(xv) + EPS
                        l = _logf(y)
                        return xv, l, l * l

                    def fbody(v, acc):
                        a0, b0, c0, a1, b1, c1 = acc
                        x0, l0, q0 = one(soff + v * 16)
                        x1, l1, q1 = one(soff + (v + HVPS) * 16)
                        return (a0 + x0, b0 + l0, c0 + q0,
                                a1 + x1, b1 + l1, c1 + q1)
                    a0, b0, c0, a1, b1, c1 = lax.fori_loop(
                        0, HVPS, fbody, (zeros,) * 6, unroll=UNROLL)
                    xt, lt, qt = one(soff + (VPS - 1) * 16)
                    vx = a0 + a1 + xt
                    vl = b0 + b1 + lt
                    vl2 = c0 + c1 + qt
                    addr = lo * 16 + laneoff
                    plsc.addupdate_scatter(ax, [addr], vx)
                    plsc.addupdate_scatter(al, [addr], vl)
                    plsc.addupdate_scatter(al2, [addr], vl2)
                    plsc.addupdate_scatter(ac, [addr], ones * float(VPS))

                @pl.when(lo != hi)
                def _():
                    @plsc.parallel_loop(0, VPS, unroll=25)
                    def vbody(v):
                        off = soff + v * 16
                        xv = xbuf[pl.ds(off, 16)]
                        iv = ibuf[pl.ds(off, 16)]
                        addr = iv * 16 + laneoff
                        y = jnp.abs(xv) + EPS
                        l = _logf(y)
                        plsc.addupdate_scatter(ax, [addr], xv)
                        plsc.addupdate_scatter(al, [addr], l)
                        plsc.addupdate_scatter(al2, [addr], l * l)
                        plsc.addupdate_scatter(ac, [addr], ones)
                return c
            lax.fori_loop(0, NSUB, sbody, 0)

        start_blk(0, xb0, ib0, sem0)

        def bbody(b2, c):
            b = b2 * 2
            start_blk(b + 1, xb1, ib1, sem1)
            wait_blk(xb0, ib0, sem0)
            process(xb0, ib0)

            @pl.when(b2 + 1 < NBLK // 2)
            def _():
                start_blk(b + 2, xb0, ib0, sem0)
            wait_blk(xb1, ib1, sem1)
            process(xb1, ib1)
            return c
        lax.fori_loop(0, NBLK // 2, bbody, 0)

        @plsc.parallel_loop(0, SEG_PAD // 16, unroll=2)
        def gbody(g):
            vx, vl, vl2, vc = zeros, zeros, zeros, zeros
            for j in range(16):
                o = (g * 16 + j) * 16
                m = laneoff == j
                vx = jnp.where(m, jnp.sum(ax[pl.ds(o, 16)]), vx)
                vl = jnp.where(m, jnp.sum(al[pl.ds(o, 16)]), vl)
                vl2 = jnp.where(m, jnp.sum(al2[pl.ds(o, 16)]), vl2)
                vc = jnp.where(m, jnp.sum(ac[pl.ds(o, 16)]), vc)
            ob[0, pl.ds(g * 16, 16)] = vx
            ob[1, pl.ds(g * 16, 16)] = vl
            ob[2, pl.ds(g * 16, 16)] = vl2
            ob[3, pl.ds(g * 16, 16)] = vc

        pltpu.sync_copy(ob, out_hbm.at[wid])

    return k(x, idx)


def _finalize(partials, tm, ts):
    def fin(p_ref, tm_ref, ts_ref, o_ref):
        s = jnp.sum(p_ref[...], axis=0)
        cnt = jnp.maximum(s[3:4, :], 1.0)
        mean = s[0:1, :] / cnt
        ml = s[1:2, :] / cnt
        var = jnp.maximum(s[2:3, :] / cnt - ml * ml, 0.0)
        std = jnp.sqrt(var + EPS)
        valid = lax.broadcasted_iota(jnp.int32, (1, SEG_PAD), 1) < NSEG
        dm = jnp.where(valid, mean - tm_ref[...], 0.0)
        dsd = jnp.where(valid, std - ts_ref[...], 0.0)
        mean_loss = jnp.sum(dm * dm) / NSEG
        std_loss = jnp.sum(dsd * dsd) / NSEG
        o_ref[0, 0] = (1.0 - STD_W) * mean_loss + STD_W * std_loss

    out = pl.pallas_call(
        fin,
        out_shape=jax.ShapeDtypeStruct((1, 1), jnp.float32),
        out_specs=pl.BlockSpec(memory_space=pltpu.SMEM),
    )(partials, tm, ts)
    return out[0, 0]


def kernel(x, idx, target_mean, target_std):
    partials = _sc_partials(x, idx)
    tm = jnp.zeros((1, SEG_PAD), jnp.float32).at[0, :NSEG].set(target_mean)
    ts = jnp.zeros((1, SEG_PAD), jnp.float32).at[0, :NSEG].set(target_std)
    return _finalize(partials, tm, ts)


# speedup vs baseline: 1.0716x; 1.0716x over previous
import functools

import jax
import jax.numpy as jnp
from jax import lax
from jax.experimental import pallas as pl
from jax.experimental.pallas import tpu as pltpu
from jax.experimental.pallas import tpu_sc as plsc

E_TOTAL = 6_400_000
NSEG = 200
SEG_PAD = 256
NC, NS = 2, 16
NW = NC * NS
CHUNK = E_TOTAL // NW
BLK = 20_000
NBLK = CHUNK // BLK
SUB = 2_000
NSUB = BLK // SUB
VPS = SUB // 16
HVPS = (VPS - 1) // 2
UNROLL = 31
EPS = 1e-6
STD_W = 0.5
LN2 = 0.6931471805599453
SQRT2 = 1.41421356237


def _logf(y):
    i = lax.bitcast_convert_type(y, jnp.int32)
    k = lax.shift_right_arithmetic(i - jnp.int32(0x3F330000), 23)
    m = lax.bitcast_convert_type(i - lax.shift_left(k, 23), jnp.float32)
    ef = k.astype(jnp.float32)
    f = m - 1.0
    z = f * f
    p = jnp.float32(0.19074033)
    for c in (-0.26126555, 0.33371909):
        p = p * f + jnp.float32(c)
    r = z * (p * f - 0.5) + f
    r = r + ef * LN2
    return r


def _sc_partials(x, idx):
    mesh = plsc.VectorSubcoreMesh(core_axis_name="c", subcore_axis_name="s")

    @functools.partial(
        pl.kernel,
        out_type=jax.ShapeDtypeStruct((NW, 4, SEG_PAD), jnp.float32),
        mesh=mesh,
        scratch_types=[
            pltpu.VMEM((BLK,), jnp.float32),
            pltpu.VMEM((BLK,), jnp.int32),
            pltpu.VMEM((BLK,), jnp.float32),
            pltpu.VMEM((BLK,), jnp.int32),
            pltpu.VMEM((16 * SEG_PAD,), jnp.float32),
            pltpu.VMEM((16 * SEG_PAD,), jnp.float32),
            pltpu.VMEM((16 * SEG_PAD,), jnp.float32),
            pltpu.VMEM((16 * SEG_PAD,), jnp.float32),
            pltpu.VMEM((4, SEG_PAD), jnp.float32),
            pltpu.SemaphoreType.DMA,
            pltpu.SemaphoreType.DMA,
        ],
        compiler_params=pltpu.CompilerParams(needs_layout_passes=False),
    )
    def k(x_hbm, idx_hbm, out_hbm, xb0, ib0, xb1, ib1, ax, al, al2, ac, ob,
          sem0, sem1):
        cid = lax.axis_index("c")
        sid = lax.axis_index("s")
        wid = sid * NC + cid
        base = wid * CHUNK
        zeros = jnp.zeros((16,), jnp.float32)
        ones = jnp.ones((16,), jnp.float32)
        laneoff = lax.iota(jnp.int32, 16)

        @plsc.parallel_loop(0, 16 * SEG_PAD // 16, unroll=8)
        def zbody(g):
            ax[pl.ds(g * 16, 16)] = zeros
            al[pl.ds(g * 16, 16)] = zeros
            al2[pl.ds(g * 16, 16)] = zeros
            ac[pl.ds(g * 16, 16)] = zeros

        def start_blk(b, xbuf, ibuf, sem):
            o = base + b * BLK
            pltpu.make_async_copy(x_hbm.at[pl.ds(o, BLK)], xbuf, sem).start()
            pltpu.make_async_copy(idx_hbm.at[pl.ds(o, BLK)], ibuf, sem).start()

        def wait_blk(xbuf, ibuf, sem):
            pltpu.make_async_copy(x_hbm.at[pl.ds(0, BLK)], xbuf, sem).wait()
            pltpu.make_async_copy(idx_hbm.at[pl.ds(0, BLK)], ibuf, sem).wait()

        def process(xbuf, ibuf):
            def sbody(sb, c):
                soff = sb * SUB
                lo = jnp.min(ibuf[pl.ds(soff, 16)])
                hi = jnp.max(ibuf[pl.ds(soff + SUB - 16, 16)])

                @pl.when(lo == hi)
                def _():
                    def one(off):
                        xv = xbuf[pl.ds(off, 16)]
                        y = jnp.abs(xv) + EPS
                        l = _logf(y)
                        return xv, l, l * l

                    def fbody(v, acc):
                        a0, b0, c0, a1, b1, c1 = acc
                        x0, l0, q0 = one(soff + v * 16)
                        x1, l1, q1 = one(soff + (v + HVPS) * 16)
                        return (a0 + x0, b0 + l0, c0 + q0,
                                a1 + x1, b1 + l1, c1 + q1)
                    a0, b0, c0, a1, b1, c1 = lax.fori_loop(
                        0, HVPS, fbody, (zeros,) * 6, unroll=UNROLL)
                    xt, lt, qt = one(soff + (VPS - 1) * 16)
                    vx = a0 + a1 + xt
                    vl = b0 + b1 + lt
                    vl2 = c0 + c1 + qt
                    addr = lo * 16 + laneoff
                    plsc.addupdate_scatter(ax, [addr], vx)
                    plsc.addupdate_scatter(al, [addr], vl)
                    plsc.addupdate_scatter(al2, [addr], vl2)
                    plsc.addupdate_scatter(ac, [addr], ones * float(VPS))

                @pl.when(lo != hi)
                def _():
                    @plsc.parallel_loop(0, VPS, unroll=25)
                    def vbody(v):
                        off = soff + v * 16
                        xv = xbuf[pl.ds(off, 16)]
                        iv = ibuf[pl.ds(off, 16)]
                        addr = iv * 16 + laneoff
                        y = jnp.abs(xv) + EPS
                        l = _logf(y)
                        plsc.addupdate_scatter(ax, [addr], xv)
                        plsc.addupdate_scatter(al, [addr], l)
                        plsc.addupdate_scatter(al2, [addr], l * l)
                        plsc.addupdate_scatter(ac, [addr], ones)
                return c
            lax.fori_loop(0, NSUB, sbody, 0)

        start_blk(0, xb0, ib0, sem0)

        def bbody(b2, c):
            b = b2 * 2
            start_blk(b + 1, xb1, ib1, sem1)
            wait_blk(xb0, ib0, sem0)
            process(xb0, ib0)

            @pl.when(b2 + 1 < NBLK // 2)
            def _():
                start_blk(b + 2, xb0, ib0, sem0)
            wait_blk(xb1, ib1, sem1)
            process(xb1, ib1)
            return c
        lax.fori_loop(0, NBLK // 2, bbody, 0)

        @plsc.parallel_loop(0, SEG_PAD // 16, unroll=2)
        def gbody(g):
            vx, vl, vl2, vc = zeros, zeros, zeros, zeros
            for j in range(16):
                o = (g * 16 + j) * 16
                m = laneoff == j
                vx = jnp.where(m, jnp.sum(ax[pl.ds(o, 16)]), vx)
                vl = jnp.where(m, jnp.sum(al[pl.ds(o, 16)]), vl)
                vl2 = jnp.where(m, jnp.sum(al2[pl.ds(o, 16)]), vl2)
                vc = jnp.where(m, jnp.sum(ac[pl.ds(o, 16)]), vc)
            ob[0, pl.ds(g * 16, 16)] = vx
            ob[1, pl.ds(g * 16, 16)] = vl
            ob[2, pl.ds(g * 16, 16)] = vl2
            ob[3, pl.ds(g * 16, 16)] = vc

        pltpu.sync_copy(ob, out_hbm.at[wid])

    return k(x, idx)


def _finalize(partials, tm, ts):
    def fin(p_ref, tm_ref, ts_ref, o_ref):
        s = jnp.sum(p_ref[...], axis=0)
        cnt = jnp.maximum(s[3:4, :], 1.0)
        mean = s[0:1, :] / cnt
        ml = s[1:2, :] / cnt
        var = jnp.maximum(s[2:3, :] / cnt - ml * ml, 0.0)
        std = jnp.sqrt(var + EPS)
        valid = lax.broadcasted_iota(jnp.int32, (1, SEG_PAD), 1) < NSEG
        dm = jnp.where(valid, mean - tm_ref[...], 0.0)
        dsd = jnp.where(valid, std - ts_ref[...], 0.0)
        mean_loss = jnp.sum(dm * dm) / NSEG
        std_loss = jnp.sum(dsd * dsd) / NSEG
        o_ref[0, 0] = (1.0 - STD_W) * mean_loss + STD_W * std_loss

    out = pl.pallas_call(
        fin,
        out_shape=jax.ShapeDtypeStruct((1, 1), jnp.float32),
        out_specs=pl.BlockSpec(memory_space=pltpu.SMEM),
    )(partials, tm, ts)
    return out[0, 0]


def kernel(x, idx, target_mean, target_std):
    partials = _sc_partials(x, idx)
    tm = jnp.zeros((1, SEG_PAD), jnp.float32).at[0, :NSEG].set(target_mean)
    ts = jnp.zeros((1, SEG_PAD), jnp.float32).at[0, :NSEG].set(target_std)
    return _finalize(partials, tm, ts)
